# Initial kernel scaffold; baseline (speedup 1.0000x reference)
import jax
import jax.numpy as jnp
from jax.experimental import pallas as pl


def kernel(voxel, points, closest_points, planes, quads):
    raise NotImplementedError("write your pallas kernel here")


# speedup vs baseline: 5.9159x; 5.9159x over previous
import functools

import jax
import jax.numpy as jnp
from jax import lax
from jax.experimental import pallas as pl
from jax.experimental.pallas import tpu as pltpu
from jax.experimental.pallas import tpu_sc as plsc

_B, _N, _G, _P = 8, 4096, 64, 8
_G3 = _G * _G * _G
_NC, _NS, _L = 2, 16, 16
_CHUNK = 1024
_GROUPS = _CHUNK // _L
_IDX_CH = 128


def _iota():
    return lax.iota(jnp.int32, _L)


def _splat_f(x):
    return jnp.full((_L,), x, jnp.float32)


def _splat_i(x):
    return jnp.full((_L,), x, jnp.int32)


def _rsqrt(s):
    i = plsc.bitcast(s, jnp.int32)
    y = plsc.bitcast(jnp.int32(0x5F3759DF) - (i >> 1), jnp.float32)
    for _ in range(3):
        y = y * (1.5 - 0.5 * s * y * y)
    return y


def _cell(z):
    zt = z.astype(jnp.int32)
    up = (z > zt.astype(jnp.float32)).astype(jnp.int32)
    return zt + up


def _lin_index(yx, yy, yz, boffv):
    ix = _cell((yx + 0.5) * _G - 0.5)
    iy = _cell((yy + 0.5) * _G - 0.5)
    iz = _cell((yz + 0.5) * _G - 0.5)
    lin = ix * (_G * _G) + iy * _G + iz
    lin = jnp.minimum(jnp.maximum(lin, 0), _G3 - 1)
    return lin + boffv


def _re_loss_vec(v_ref, row0):
    it = _iota()
    lmask = it < 7
    acc = jnp.zeros((_L,), jnp.float32)
    for b in range(_B):
        cols = []
        s = jnp.zeros((_L,), jnp.float32)
        for i in range(4):
            idx = jnp.minimum(b * 32 + (row0 + it) * 4 + i, _B * 32 - 1)
            cl = plsc.load_gather(v_ref, [idx])
            cl = jnp.where(lmask, cl, 0.0)
            cols.append(cl)
            s = s + cl * cl
        r = _rsqrt(s)
        cols = [cl * r for cl in cols]
        for i in range(4):
            for j in range(4):
                m = jnp.sum(cols[i] * cols[j])
                d = _splat_f(m) - (1.0 if i == j else 0.0)
                acc = acc + d * d
    return acc


def _sc_body(pts_hbm, planes_hbm, quads_hbm, cpx_hbm, cpy_hbm, cpz_hbm,
             vox_hbm, out_hbm,
             pts_v, planes_v, quads_v,
             ypx_v, ypy_v, ypz_v, yqx_v, yqy_v, yqz_v,
             idxp_v, idxq_v,
             cxp_v, cyp_v, czp_v, voxp_v,
             cxq_v, cyq_v, czq_v, voxq_v,
             st3_v, mst_v, lst_v, ost_v,
             sh_mid, sh_p, sem):
    cid = lax.axis_index("c")
    sid = lax.axis_index("s")
    b = cid * 4 + sid // 4
    chunk = sid % 4

    pltpu.sync_copy(pts_hbm.at[pl.ds((b * _N + chunk * _CHUNK) * 3, _CHUNK * 3)],
                    pts_v)
    pltpu.sync_copy(planes_hbm, planes_v)
    pltpu.sync_copy(quads_hbm, quads_v)

    it = _iota()
    z16 = jnp.zeros((_L,), jnp.float32)

    def gxyz(i):
        ii = it * 3 + i * (3 * _L)
        return (plsc.load_gather(pts_v, [ii]),
                plsc.load_gather(pts_v, [ii + 1]),
                plsc.load_gather(pts_v, [ii + 2]))

    def mid_body(i, acc):
        ax, ay, az = acc
        vx, vy, vz = gxyz(i)
        return (ax + vx, ay + vy, az + vz)

    ax, ay, az = lax.fori_loop(0, _GROUPS, mid_body, (z16, z16, z16))
    st3_v[pl.ds(0, _L)] = ax
    st3_v[pl.ds(_L, _L)] = ay
    st3_v[pl.ds(2 * _L, _L)] = az
    pltpu.sync_copy(st3_v, sh_mid.at[pl.ds(sid * 3 * _L, 3 * _L)])
    plsc.subcore_barrier()
    pltpu.sync_copy(sh_mid.at[pl.ds((sid // 4) * 4 * 3 * _L, 4 * 3 * _L)], mst_v)
    mx, my, mz = z16, z16, z16
    for k in range(4):
        mx = mx + mst_v[pl.ds(k * 3 * _L, _L)]
        my = my + mst_v[pl.ds(k * 3 * _L + _L, _L)]
        mz = mz + mst_v[pl.ds(k * 3 * _L + 2 * _L, _L)]
    midx = _splat_f(jnp.sum(mx)) * (1.0 / _N)
    midy = _splat_f(jnp.sum(my)) * (1.0 / _N)
    midz = _splat_f(jnp.sum(mz)) * (1.0 / _N)

    boffv = _splat_i(b * _G3)
    for t in range(2):
        pb = (b * _P + chunk * 2 + t) * 4
        pvec = jnp.minimum(pb + it, _B * _P * 4 - 1)
        prow = plsc.load_gather(planes_v, [pvec])
        qrow = plsc.load_gather(quads_v, [pvec])
        pn0 = _splat_f(prow[0])
        pn1 = _splat_f(prow[1])
        pn2 = _splat_f(prow[2])
        pdd = _splat_f(prow[3])
        rinv = _rsqrt(pn0 * pn0 + pn1 * pn1 + pn2 * pn2)
        n0 = pn0 * rinv
        n1 = pn1 * rinv
        n2 = pn2 * rinv
        dd = pdd * rinv

        q1 = _splat_f(qrow[1])
        q2 = _splat_f(qrow[2])
        q3 = _splat_f(qrow[3])
        qr = _rsqrt(q1 * q1 + q2 * q2 + q3 * q3) * 0.707
        ux = q1 * qr
        uy = q2 * qr
        uz = q3 * qr
        qw = 0.707
        w2mu = qw * qw - (ux * ux + uy * uy + uz * uz)

        def tr_body(g, _, t=t, n0=n0, n1=n1, n2=n2, dd=dd,
                    ux=ux, uy=uy, uz=uz, w2mu=w2mu):
            i = t * (_GROUPS // 2) + g
            vx, vy, vz = gxyz(i)
            dot = vx * n0 + vy * n1 + vz * n2 + dd
            px = vx - 2.0 * dot * n0
            py = vy - 2.0 * dot * n1
            pz = vz - 2.0 * dot * n2
            off = i * _L
            ypx_v[pl.ds(off, _L)] = px
            ypy_v[pl.ds(off, _L)] = py
            ypz_v[pl.ds(off, _L)] = pz
            idxp_v[pl.ds(off, _L)] = _lin_index(px, py, pz, boffv)
            wx = vx - midx
            wy = vy - midy
            wz = vz - midz
            uv = ux * wx + uy * wy + uz * wz
            cx = uy * wz - uz * wy
            cy = uz * wx - ux * wz
            cz = ux * wy - uy * wx
            qx = w2mu * wx + 2.0 * uv * ux + (2.0 * qw) * cx
            qy = w2mu * wy + 2.0 * uv * uy + (2.0 * qw) * cy
            qz_ = w2mu * wz + 2.0 * uv * uz + (2.0 * qw) * cz
            yqx_v[pl.ds(off, _L)] = qx
            yqy_v[pl.ds(off, _L)] = qy
            yqz_v[pl.ds(off, _L)] = qz_
            idxq_v[pl.ds(off, _L)] = _lin_index(qx, qy, qz_, boffv)
            return 0

        lax.fori_loop(0, _GROUPS // 2, tr_body, 0)

    copies = []
    for k in range(_CHUNK // _IDX_CH):
        s0 = k * _IDX_CH
        for idx_v, dsts in ((idxp_v, (cxp_v, cyp_v, czp_v, voxp_v)),
                            (idxq_v, (cxq_v, cyq_v, czq_v, voxq_v))):
            isl = idx_v.at[pl.ds(s0, _IDX_CH)]
            for tab, dst in zip((cpx_hbm, cpy_hbm, cpz_hbm, vox_hbm), dsts):
                copies.append(pltpu.async_copy(
                    tab.at[isl], dst.at[pl.ds(s0, _IDX_CH)], sem))
    for cph in copies:
        cph.wait()

    def loss_body(i, acc):
        aP, aQ = acc
        off = i * _L
        mP = 1.0 - voxp_v[pl.ds(off, _L)]
        dx = (ypx_v[pl.ds(off, _L)] - cxp_v[pl.ds(off, _L)]) * mP
        dy = (ypy_v[pl.ds(off, _L)] - cyp_v[pl.ds(off, _L)]) * mP
        dz = (ypz_v[pl.ds(off, _L)] - czp_v[pl.ds(off, _L)]) * mP
        aP = aP + dx * dx + dy * dy + dz * dz
        mQ = 1.0 - voxq_v[pl.ds(off, _L)]
        ex = (yqx_v[pl.ds(off, _L)] - cxq_v[pl.ds(off, _L)]) * mQ
        ey = (yqy_v[pl.ds(off, _L)] - cyq_v[pl.ds(off, _L)]) * mQ
        ez = (yqz_v[pl.ds(off, _L)] - czq_v[pl.ds(off, _L)]) * mQ
        aQ = aQ + ex * ex + ey * ey + ez * ez
        return (aP, aQ)

    accP, accQ = lax.fori_loop(0, _GROUPS, loss_body, (z16, z16))

    st3_v[pl.ds(0, _L)] = accP
    st3_v[pl.ds(_L, _L)] = accQ
    pltpu.sync_copy(st3_v.at[pl.ds(0, 2 * _L)],
                    sh_p.at[pl.ds(sid * 2 * _L, 2 * _L)])
    plsc.subcore_barrier()

    @pl.when(sid == 0)
    def _():
        pltpu.sync_copy(sh_p, lst_v)
        tp = jnp.zeros((_L,), jnp.float32)
        tq = jnp.zeros((_L,), jnp.float32)
        for k in range(_NS):
            tp = tp + lst_v[pl.ds(k * 2 * _L, _L)]
            tq = tq + lst_v[pl.ds(k * 2 * _L + _L, _L)]
        totP = jnp.sum(tp)
        totQ = jnp.sum(tq)

        reP = _re_loss_vec(planes_v, 0)
        reQ = _re_loss_vec(quads_v, 1)
        gate = jnp.where(_splat_i(cid) == 0, 1.0, 0.0)

        val = (_splat_f(totP) * (50.0 / _P)
               + _splat_f(totQ) * (50.0 / (_P * _N))
               + (reP + reQ) * gate)
        ost_v[:] = jnp.where(_iota() == 0, val, 0.0)
        pltpu.sync_copy(ost_v, out_hbm.at[cid])


@jax.jit
def _prs_loss_sc(pts_flat, planes_flat, quads_flat, cpx_tab, cpy_tab,
                 cpz_tab, vox_tab):
    mesh = plsc.VectorSubcoreMesh(core_axis_name="c", subcore_axis_name="s",
                                  num_cores=_NC, num_subcores=_NS)
    f32 = jnp.float32
    i32 = jnp.int32
    run = pl.kernel(
        _sc_body,
        out_type=jax.ShapeDtypeStruct((_NC, _L), f32),
        mesh=mesh,
        compiler_params=pltpu.CompilerParams(needs_layout_passes=False,
                                             use_tc_tiling_on_sc=False),
        scratch_types=[
            pltpu.VMEM((_CHUNK * 3,), f32),
            pltpu.VMEM((_B * _P * 4,), f32),
            pltpu.VMEM((_B * _P * 4,), f32),
            pltpu.VMEM((_CHUNK,), f32),
            pltpu.VMEM((_CHUNK,), f32),
            pltpu.VMEM((_CHUNK,), f32),
            pltpu.VMEM((_CHUNK,), f32),
            pltpu.VMEM((_CHUNK,), f32),
            pltpu.VMEM((_CHUNK,), f32),
            pltpu.VMEM((_CHUNK,), i32),
            pltpu.VMEM((_CHUNK,), i32),
            pltpu.VMEM((_CHUNK,), f32),
            pltpu.VMEM((_CHUNK,), f32),
            pltpu.VMEM((_CHUNK,), f32),
            pltpu.VMEM((_CHUNK,), f32),
            pltpu.VMEM((_CHUNK,), f32),
            pltpu.VMEM((_CHUNK,), f32),
            pltpu.VMEM((_CHUNK,), f32),
            pltpu.VMEM((_CHUNK,), f32),
            pltpu.VMEM((3 * _L,), f32),
            pltpu.VMEM((4 * 3 * _L,), f32),
            pltpu.VMEM((_NS * 2 * _L,), f32),
            pltpu.VMEM((_L,), f32),
            pltpu.VMEM_SHARED((_NS * 3 * _L,), f32),
            pltpu.VMEM_SHARED((_NS * 2 * _L,), f32),
            pltpu.SemaphoreType.DMA,
        ],
    )
    return run(pts_flat, planes_flat, quads_flat, cpx_tab, cpy_tab,
               cpz_tab, vox_tab)


def kernel(voxel, points, closest_points, planes, quads):
    out = _prs_loss_sc(points.reshape(-1),
                       planes.reshape(-1),
                       quads.reshape(-1),
                       closest_points[..., 0].reshape(-1),
                       closest_points[..., 1].reshape(-1),
                       closest_points[..., 2].reshape(-1),
                       voxel.reshape(-1))
    return out[0, 0] + out[1, 0]
